# Initial kernel scaffold; baseline (speedup 1.0000x reference)
import functools

import jax
import jax.numpy as jnp
from jax import lax
from jax.experimental import pallas as pl
from jax.experimental.pallas import tpu as pltpu
from jax.experimental.pallas import tpu_sc as plsc

N = 10000
NP = 10240
E = 320000
NC = 2
NS = 16
NW = NC * NS
EW = E // NW
CH = 80
NCHUNK = EW // CH
RPT = NP // NS

_MESH = plsc.VectorSubcoreMesh(core_axis_name="c", subcore_axis_name="s")


@functools.cache
def _make_edge_scatter(F):
    @functools.partial(
        pl.kernel,
        out_type=jax.ShapeDtypeStruct((NC, NP, F), jnp.float32),
        mesh=_MESH,
        scratch_types=[
            pltpu.VMEM((NCHUNK, CH), jnp.int32),
            pltpu.VMEM((NCHUNK, CH), jnp.int32),
            pltpu.VMEM((CH, F), jnp.float32),
            pltpu.VMEM_SHARED((NP, F), jnp.float32),
            pltpu.SemaphoreType.DMA,
        ],
    )
    def k(g_hbm, row_hbm, col_hbm, zero_hbm, out_hbm, idx_r, idx_c, buf, acc, sem):
        cid = lax.axis_index("c")
        sid = lax.axis_index("s")
        wid = cid * NS + sid
        r0 = sid * RPT
        pltpu.sync_copy(zero_hbm.at[pl.ds(r0, RPT)], acc.at[pl.ds(r0, RPT)])
        pltpu.sync_copy(row_hbm.at[wid], idx_r)
        pltpu.sync_copy(col_hbm.at[wid], idx_c)
        plsc.subcore_barrier()

        def body(j, carry):
            pltpu.async_copy(g_hbm.at[idx_r.at[j]], buf, sem).wait()
            pltpu.sync_copy(buf, acc.at[idx_c.at[j]], add=True)
            return carry

        lax.fori_loop(0, NCHUNK, body, 0)
        plsc.subcore_barrier()
        pltpu.sync_copy(acc.at[pl.ds(r0, RPT)], out_hbm.at[cid, pl.ds(r0, RPT)])

    return k


@functools.partial(
    pl.kernel,
    out_type=jax.ShapeDtypeStruct((NC, NP), jnp.float32),
    mesh=_MESH,
    scratch_types=[
        pltpu.VMEM((NCHUNK, CH), jnp.int32),
        pltpu.VMEM((CH,), jnp.float32),
        pltpu.VMEM_SHARED((NP,), jnp.float32),
    ],
)
def _deg_kernel(col_hbm, zero_hbm, out_hbm, idx_c, ones_v, acc):
    cid = lax.axis_index("c")
    sid = lax.axis_index("s")
    wid = cid * NS + sid
    r0 = sid * RPT
    pltpu.sync_copy(zero_hbm.at[pl.ds(r0, RPT)], acc.at[pl.ds(r0, RPT)])
    pltpu.sync_copy(col_hbm.at[wid], idx_c)
    for i in range(CH // 16):
        ones_v[pl.ds(i * 16, 16)] = jnp.ones((16,), jnp.float32)
    plsc.subcore_barrier()

    def body(j, carry):
        pltpu.sync_copy(ones_v, acc.at[idx_c.at[j]], add=True)
        return carry

    lax.fori_loop(0, NCHUNK, body, 0)
    plsc.subcore_barrier()
    pltpu.sync_copy(acc.at[pl.ds(r0, RPT)], out_hbm.at[cid, pl.ds(r0, RPT)])


BR = 1000
GRID = (N // BR,)


def _row(f):
    return pl.BlockSpec((BR, f), lambda i: (i, 0))


def _full(a, b):
    return pl.BlockSpec((a, b), lambda i: (0, 0))


def _f32(*shape):
    return jax.ShapeDtypeStruct(shape, jnp.float32)


def _k1_body(deg0, deg1, x, w, dinv_o, g_o):
    dinv = lax.rsqrt(deg0[...] + deg1[...] + 1.0)
    dinv_o[...] = dinv
    g_o[...] = dinv * jnp.dot(x[...], w[...], preferred_element_type=jnp.float32)


_k1 = pl.pallas_call(
    _k1_body,
    grid=GRID,
    in_specs=[_row(1), _row(1), _row(128), _full(128, 64)],
    out_specs=[_row(1), _row(64)],
    out_shape=[_f32(N, 1), _f32(N, 64)],
)


def _k2_body(s0, s1, g, dinv, b, w, g2_o):
    dv = dinv[...]
    h = jnp.maximum(dv * (s0[...] + s1[...] + g[...]) + b[...], 0.0)
    g2_o[...] = dv * jnp.dot(h, w[...], preferred_element_type=jnp.float32)


_k2 = pl.pallas_call(
    _k2_body,
    grid=GRID,
    in_specs=[_row(64), _row(64), _row(64), _row(1), _full(1, 64), _full(64, 32)],
    out_specs=[_row(32)],
    out_shape=[_f32(N, 32)],
)


def _k3_body(s0, s1, g, dinv, b, wmu, bmu, wlv, blv, wd1, mu_o, lv_o, g3_o):
    dv = dinv[...]
    h2 = dv * (s0[...] + s1[...] + g[...]) + b[...]
    mu = jnp.dot(h2, wmu[...], preferred_element_type=jnp.float32) + bmu[...]
    lv = jnp.dot(h2, wlv[...], preferred_element_type=jnp.float32) + blv[...]
    mu_o[...] = mu
    lv_o[...] = lv
    g3_o[...] = dv * jnp.dot(mu, wd1[...], preferred_element_type=jnp.float32)


_k3 = pl.pallas_call(
    _k3_body,
    grid=GRID,
    in_specs=[
        _row(32), _row(32), _row(32), _row(1), _full(1, 32),
        _full(32, 32), _full(1, 32), _full(32, 32), _full(1, 32), _full(32, 128),
    ],
    out_specs=[_row(32), _row(32), _row(128)],
    out_shape=[_f32(N, 32), _f32(N, 32), _f32(N, 128)],
)


def _k4_body(s0, s1, g, dinv, b, w, g4_o):
    dv = dinv[...]
    d = jnp.maximum(dv * (s0[...] + s1[...] + g[...]) + b[...], 0.0)
    g4_o[...] = dv * jnp.dot(d, w[...], preferred_element_type=jnp.float32)


_k4 = pl.pallas_call(
    _k4_body,
    grid=GRID,
    in_specs=[_row(128), _row(128), _row(128), _row(1), _full(1, 128), _full(128, 128)],
    out_specs=[_row(128)],
    out_shape=[_f32(N, 128)],
)


def _k5_body(s0, s1, g, dinv, b, recon_o):
    recon_o[...] = dinv[...] * (s0[...] + s1[...] + g[...]) + b[...]


_k5 = pl.pallas_call(
    _k5_body,
    grid=GRID,
    in_specs=[_row(128), _row(128), _row(128), _row(1), _full(1, 128)],
    out_specs=[_row(128)],
    out_shape=[_f32(N, 128)],
)


def kernel(x, edge_index, W_e1, b_e1, W_e2, b_e2, W_mu, b_mu, W_lv, b_lv,
           W_d1, b_d1, W_d2, b_d2):
    ei = edge_index.astype(jnp.int32)
    row = ei[0].reshape(NW, NCHUNK, CH)
    col = ei[1].reshape(NW, NCHUNK, CH)

    degp = _deg_kernel(col, jnp.zeros((NP,), jnp.float32))
    deg0 = degp[0, :N].reshape(N, 1)
    deg1 = degp[1, :N].reshape(N, 1)

    dinv, g1 = _k1(deg0, deg1, x, W_e1)

    s1 = _make_edge_scatter(64)(g1, row, col, jnp.zeros((NP, 64), jnp.float32))
    g2 = _k2(s1[0, :N], s1[1, :N], g1, dinv, b_e1.reshape(1, 64), W_e2)

    s2 = _make_edge_scatter(32)(g2, row, col, jnp.zeros((NP, 32), jnp.float32))
    mu, logvar, g3 = _k3(
        s2[0, :N], s2[1, :N], g2, dinv, b_e2.reshape(1, 32),
        W_mu, b_mu.reshape(1, 32), W_lv, b_lv.reshape(1, 32), W_d1,
    )

    s3 = _make_edge_scatter(128)(g3, row, col, jnp.zeros((NP, 128), jnp.float32))
    g4 = _k4(s3[0, :N], s3[1, :N], g3, dinv, b_d1.reshape(1, 128), W_d2)

    s4 = _make_edge_scatter(128)(g4, row, col, jnp.zeros((NP, 128), jnp.float32))
    recon = _k5(s4[0, :N], s4[1, :N], g4, dinv, b_d2.reshape(1, 128))

    return recon, mu, logvar


# speedup vs baseline: 16.4698x; 16.4698x over previous
import functools

import jax
import jax.numpy as jnp
from jax import lax
from jax.experimental import pallas as pl
from jax.experimental.pallas import tpu as pltpu
from jax.experimental.pallas import tpu_sc as plsc

N = 10000
NP = 10240
E = 320000
NC = 2
NS = 16
NW = NC * NS
EW = E // NW
CH = 80
NCHUNK = EW // CH
RPT = NP // NS

_MESH = plsc.VectorSubcoreMesh(core_axis_name="c", subcore_axis_name="s")


@functools.cache
def _make_edge_scatter(F):
    @functools.partial(
        pl.kernel,
        out_type=jax.ShapeDtypeStruct((NC, NP, F), jnp.float32),
        mesh=_MESH,
        scratch_types=[
            pltpu.VMEM((NCHUNK, CH), jnp.int32),
            pltpu.VMEM((NCHUNK, CH), jnp.int32),
            pltpu.VMEM((CH, F), jnp.float32),
            pltpu.VMEM_SHARED((NP, F), jnp.float32),
            pltpu.SemaphoreType.DMA,
        ],
        compiler_params=pltpu.CompilerParams(use_tc_tiling_on_sc=False),
    )
    def k(g_hbm, row_hbm, col_hbm, zero_hbm, out_hbm, idx_r, idx_c, buf, acc, sem):
        cid = lax.axis_index("c")
        sid = lax.axis_index("s")
        wid = cid * NS + sid
        r0 = sid * RPT
        pltpu.sync_copy(zero_hbm.at[pl.ds(r0, RPT)], acc.at[pl.ds(r0, RPT)])
        pltpu.sync_copy(row_hbm.at[wid], idx_r)
        pltpu.sync_copy(col_hbm.at[wid], idx_c)
        plsc.subcore_barrier()

        def body(j, carry):
            pltpu.async_copy(g_hbm.at[idx_r.at[j]], buf, sem).wait()
            pltpu.sync_copy(buf, acc.at[idx_c.at[j]], add=True)
            return carry

        lax.fori_loop(0, NCHUNK, body, 0)
        plsc.subcore_barrier()
        pltpu.sync_copy(acc.at[pl.ds(r0, RPT)], out_hbm.at[cid, pl.ds(r0, RPT)])

    return k


@functools.partial(
    pl.kernel,
    out_type=jax.ShapeDtypeStruct((NC, NP), jnp.float32),
    mesh=_MESH,
    scratch_types=[
        pltpu.VMEM((NCHUNK, CH), jnp.int32),
        pltpu.VMEM((CH,), jnp.float32),
        pltpu.VMEM_SHARED((NP,), jnp.float32),
    ],
)
def _deg_kernel(col_hbm, zero_hbm, out_hbm, idx_c, ones_v, acc):
    cid = lax.axis_index("c")
    sid = lax.axis_index("s")
    wid = cid * NS + sid
    r0 = sid * RPT
    pltpu.sync_copy(zero_hbm.at[pl.ds(r0, RPT)], acc.at[pl.ds(r0, RPT)])
    pltpu.sync_copy(col_hbm.at[wid], idx_c)
    for i in range(CH // 16):
        ones_v[pl.ds(i * 16, 16)] = jnp.ones((16,), jnp.float32)
    plsc.subcore_barrier()

    def body(j, carry):
        pltpu.sync_copy(ones_v, acc.at[idx_c.at[j]], add=True)
        return carry

    lax.fori_loop(0, NCHUNK, body, 0)
    plsc.subcore_barrier()
    pltpu.sync_copy(acc.at[pl.ds(r0, RPT)], out_hbm.at[cid, pl.ds(r0, RPT)])


BR = 1000
GRID = (N // BR,)


def _row(f):
    return pl.BlockSpec((BR, f), lambda i: (i, 0))


def _full(a, b):
    return pl.BlockSpec((a, b), lambda i: (0, 0))


def _f32(*shape):
    return jax.ShapeDtypeStruct(shape, jnp.float32)


def _k1_body(deg0, deg1, x, w, dinv_o, g_o):
    dinv = lax.rsqrt(deg0[...] + deg1[...] + 1.0)
    dinv_o[...] = dinv
    g_o[...] = dinv * jnp.dot(x[...], w[...], preferred_element_type=jnp.float32)


_k1 = pl.pallas_call(
    _k1_body,
    grid=GRID,
    in_specs=[_row(1), _row(1), _row(128), _full(128, 64)],
    out_specs=[_row(1), _row(64)],
    out_shape=[_f32(N, 1), _f32(N, 64)],
)


def _k2_body(s0, s1, g, dinv, b, w, g2_o):
    dv = dinv[...]
    h = jnp.maximum(dv * (s0[...] + s1[...] + g[...]) + b[...], 0.0)
    g2_o[...] = dv * jnp.dot(h, w[...], preferred_element_type=jnp.float32)


_k2 = pl.pallas_call(
    _k2_body,
    grid=GRID,
    in_specs=[_row(64), _row(64), _row(64), _row(1), _full(1, 64), _full(64, 32)],
    out_specs=[_row(32)],
    out_shape=[_f32(N, 32)],
)


def _k3_body(s0, s1, g, dinv, b, wmu, bmu, wlv, blv, wd1, mu_o, lv_o, g3_o):
    dv = dinv[...]
    h2 = dv * (s0[...] + s1[...] + g[...]) + b[...]
    mu = jnp.dot(h2, wmu[...], preferred_element_type=jnp.float32) + bmu[...]
    lv = jnp.dot(h2, wlv[...], preferred_element_type=jnp.float32) + blv[...]
    mu_o[...] = mu
    lv_o[...] = lv
    g3_o[...] = dv * jnp.dot(mu, wd1[...], preferred_element_type=jnp.float32)


_k3 = pl.pallas_call(
    _k3_body,
    grid=GRID,
    in_specs=[
        _row(32), _row(32), _row(32), _row(1), _full(1, 32),
        _full(32, 32), _full(1, 32), _full(32, 32), _full(1, 32), _full(32, 128),
    ],
    out_specs=[_row(32), _row(32), _row(128)],
    out_shape=[_f32(N, 32), _f32(N, 32), _f32(N, 128)],
)


def _k4_body(s0, s1, g, dinv, b, w, g4_o):
    dv = dinv[...]
    d = jnp.maximum(dv * (s0[...] + s1[...] + g[...]) + b[...], 0.0)
    g4_o[...] = dv * jnp.dot(d, w[...], preferred_element_type=jnp.float32)


_k4 = pl.pallas_call(
    _k4_body,
    grid=GRID,
    in_specs=[_row(128), _row(128), _row(128), _row(1), _full(1, 128), _full(128, 128)],
    out_specs=[_row(128)],
    out_shape=[_f32(N, 128)],
)


def _k5_body(s0, s1, g, dinv, b, recon_o):
    recon_o[...] = dinv[...] * (s0[...] + s1[...] + g[...]) + b[...]


_k5 = pl.pallas_call(
    _k5_body,
    grid=GRID,
    in_specs=[_row(128), _row(128), _row(128), _row(1), _full(1, 128)],
    out_specs=[_row(128)],
    out_shape=[_f32(N, 128)],
)


def kernel(x, edge_index, W_e1, b_e1, W_e2, b_e2, W_mu, b_mu, W_lv, b_lv,
           W_d1, b_d1, W_d2, b_d2):
    ei = edge_index.astype(jnp.int32)
    row = ei[0].reshape(NW, NCHUNK, CH)
    col = ei[1].reshape(NW, NCHUNK, CH)

    degp = _deg_kernel(col, jnp.zeros((NP,), jnp.float32))
    deg0 = degp[0, :N].reshape(N, 1)
    deg1 = degp[1, :N].reshape(N, 1)

    dinv, g1 = _k1(deg0, deg1, x, W_e1)

    s1 = _make_edge_scatter(64)(g1, row, col, jnp.zeros((NP, 64), jnp.float32))
    (g2,) = _k2(s1[0, :N], s1[1, :N], g1, dinv, b_e1.reshape(1, 64), W_e2)

    s2 = _make_edge_scatter(32)(g2, row, col, jnp.zeros((NP, 32), jnp.float32))
    mu, logvar, g3 = _k3(
        s2[0, :N], s2[1, :N], g2, dinv, b_e2.reshape(1, 32),
        W_mu, b_mu.reshape(1, 32), W_lv, b_lv.reshape(1, 32), W_d1,
    )

    s3 = _make_edge_scatter(128)(g3, row, col, jnp.zeros((NP, 128), jnp.float32))
    (g4,) = _k4(s3[0, :N], s3[1, :N], g3, dinv, b_d1.reshape(1, 128), W_d2)

    s4 = _make_edge_scatter(128)(g4, row, col, jnp.zeros((NP, 128), jnp.float32))
    (recon,) = _k5(s4[0, :N], s4[1, :N], g4, dinv, b_d2.reshape(1, 128))

    return recon, mu, logvar


# speedup vs baseline: 19.9745x; 1.2128x over previous
import functools

import jax
import jax.numpy as jnp
from jax import lax
from jax.experimental import pallas as pl
from jax.experimental.pallas import tpu as pltpu
from jax.experimental.pallas import tpu_sc as plsc

N = 10000
NP = 10240
E = 320000
NC = 2
NS = 16
NW = NC * NS
EW = E // NW
CH = 80
NCHUNK = EW // CH
NBUF = 5
RPT = NP // NS

_MESH = plsc.VectorSubcoreMesh(core_axis_name="c", subcore_axis_name="s")


@functools.cache
def _make_edge_scatter(F):
    @functools.partial(
        pl.kernel,
        out_type=jax.ShapeDtypeStruct((NC, NP, F), jnp.float32),
        mesh=_MESH,
        scratch_types=[
            pltpu.VMEM((NCHUNK, CH), jnp.int32),
            pltpu.VMEM((NCHUNK, CH), jnp.int32),
            pltpu.VMEM((CH, F), jnp.float32),
            pltpu.VMEM((CH, F), jnp.float32),
            pltpu.VMEM_SHARED((NP, F), jnp.float32),
            pltpu.SemaphoreType.DMA,
            pltpu.SemaphoreType.DMA,
        ],
        compiler_params=pltpu.CompilerParams(use_tc_tiling_on_sc=False),
    )
    def k(g_hbm, row_hbm, col_hbm, zero_hbm, out_hbm, idx_r, idx_c,
          buf0, buf1, acc, sem0, sem1):
        cid = lax.axis_index("c")
        sid = lax.axis_index("s")
        wid = cid * NS + sid
        r0 = sid * RPT
        pltpu.sync_copy(zero_hbm.at[pl.ds(r0, RPT)], acc.at[pl.ds(r0, RPT)])
        pltpu.sync_copy(row_hbm.at[wid], idx_r)
        pltpu.sync_copy(col_hbm.at[wid], idx_c)
        plsc.subcore_barrier()

        pltpu.async_copy(g_hbm.at[idx_r.at[0]], buf0, sem0)

        def body(k_, carry):
            j = 2 * k_
            pltpu.make_async_copy(g_hbm.at[idx_r.at[j]], buf0, sem0).wait()
            pltpu.async_copy(g_hbm.at[idx_r.at[j + 1]], buf1, sem1)
            pltpu.sync_copy(buf0, acc.at[idx_c.at[j]], add=True)
            pltpu.make_async_copy(g_hbm.at[idx_r.at[j + 1]], buf1, sem1).wait()
            pltpu.async_copy(g_hbm.at[idx_r.at[j + 2]], buf0, sem0)
            pltpu.sync_copy(buf1, acc.at[idx_c.at[j + 1]], add=True)
            return carry

        lax.fori_loop(0, (NCHUNK - 1) // 2, body, 0)
        pltpu.make_async_copy(g_hbm.at[idx_r.at[NCHUNK - 1]], buf0, sem0).wait()
        pltpu.sync_copy(buf0, acc.at[idx_c.at[NCHUNK - 1]], add=True)
        plsc.subcore_barrier()
        pltpu.sync_copy(acc.at[pl.ds(r0, RPT)], out_hbm.at[cid, pl.ds(r0, RPT)])

    return k


@functools.partial(
    pl.kernel,
    out_type=jax.ShapeDtypeStruct((NC, NP), jnp.float32),
    mesh=_MESH,
    scratch_types=[
        pltpu.VMEM((NCHUNK, CH), jnp.int32),
        pltpu.VMEM((CH,), jnp.float32),
        pltpu.VMEM_SHARED((NP,), jnp.float32),
    ],
)
def _deg_kernel(col_hbm, zero_hbm, out_hbm, idx_c, ones_v, acc):
    cid = lax.axis_index("c")
    sid = lax.axis_index("s")
    wid = cid * NS + sid
    r0 = sid * RPT
    pltpu.sync_copy(zero_hbm.at[pl.ds(r0, RPT)], acc.at[pl.ds(r0, RPT)])
    pltpu.sync_copy(col_hbm.at[wid], idx_c)
    for i in range(CH // 16):
        ones_v[pl.ds(i * 16, 16)] = jnp.ones((16,), jnp.float32)
    plsc.subcore_barrier()

    def body(j, carry):
        pltpu.sync_copy(ones_v, acc.at[idx_c.at[j]], add=True)
        return carry

    lax.fori_loop(0, NCHUNK, body, 0)
    plsc.subcore_barrier()
    pltpu.sync_copy(acc.at[pl.ds(r0, RPT)], out_hbm.at[cid, pl.ds(r0, RPT)])


BR = 1000
GRID = (N // BR,)


def _row(f):
    return pl.BlockSpec((BR, f), lambda i: (i, 0))


def _full(a, b):
    return pl.BlockSpec((a, b), lambda i: (0, 0))


def _f32(*shape):
    return jax.ShapeDtypeStruct(shape, jnp.float32)


def _k1_body(deg0, deg1, x, w, dinv_o, g_o):
    dinv = lax.rsqrt(deg0[...] + deg1[...] + 1.0)
    dinv_o[...] = dinv
    g_o[...] = dinv * jnp.dot(x[...], w[...], preferred_element_type=jnp.float32)


_k1 = pl.pallas_call(
    _k1_body,
    grid=GRID,
    in_specs=[_row(1), _row(1), _row(128), _full(128, 64)],
    out_specs=[_row(1), _row(64)],
    out_shape=[_f32(N, 1), _f32(N, 64)],
)


def _k2_body(s0, s1, g, dinv, b, w, g2_o):
    dv = dinv[...]
    h = jnp.maximum(dv * (s0[...] + s1[...] + g[...]) + b[...], 0.0)
    g2_o[...] = dv * jnp.dot(h, w[...], preferred_element_type=jnp.float32)


_k2 = pl.pallas_call(
    _k2_body,
    grid=GRID,
    in_specs=[_row(64), _row(64), _row(64), _row(1), _full(1, 64), _full(64, 32)],
    out_specs=[_row(32)],
    out_shape=[_f32(N, 32)],
)


def _k3_body(s0, s1, g, dinv, b, wmu, bmu, wlv, blv, wd1, mu_o, lv_o, g3_o):
    dv = dinv[...]
    h2 = dv * (s0[...] + s1[...] + g[...]) + b[...]
    mu = jnp.dot(h2, wmu[...], preferred_element_type=jnp.float32) + bmu[...]
    lv = jnp.dot(h2, wlv[...], preferred_element_type=jnp.float32) + blv[...]
    mu_o[...] = mu
    lv_o[...] = lv
    g3_o[...] = dv * jnp.dot(mu, wd1[...], preferred_element_type=jnp.float32)


_k3 = pl.pallas_call(
    _k3_body,
    grid=GRID,
    in_specs=[
        _row(32), _row(32), _row(32), _row(1), _full(1, 32),
        _full(32, 32), _full(1, 32), _full(32, 32), _full(1, 32), _full(32, 128),
    ],
    out_specs=[_row(32), _row(32), _row(128)],
    out_shape=[_f32(N, 32), _f32(N, 32), _f32(N, 128)],
)


def _k4_body(s0, s1, g, dinv, b, w, g4_o):
    dv = dinv[...]
    d = jnp.maximum(dv * (s0[...] + s1[...] + g[...]) + b[...], 0.0)
    g4_o[...] = dv * jnp.dot(d, w[...], preferred_element_type=jnp.float32)


_k4 = pl.pallas_call(
    _k4_body,
    grid=GRID,
    in_specs=[_row(128), _row(128), _row(128), _row(1), _full(1, 128), _full(128, 128)],
    out_specs=[_row(128)],
    out_shape=[_f32(N, 128)],
)


def _k5_body(s0, s1, g, dinv, b, recon_o):
    recon_o[...] = dinv[...] * (s0[...] + s1[...] + g[...]) + b[...]


_k5 = pl.pallas_call(
    _k5_body,
    grid=GRID,
    in_specs=[_row(128), _row(128), _row(128), _row(1), _full(1, 128)],
    out_specs=[_row(128)],
    out_shape=[_f32(N, 128)],
)


def kernel(x, edge_index, W_e1, b_e1, W_e2, b_e2, W_mu, b_mu, W_lv, b_lv,
           W_d1, b_d1, W_d2, b_d2):
    ei = edge_index.astype(jnp.int32)
    row = ei[0].reshape(NW, NCHUNK, CH)
    col = ei[1].reshape(NW, NCHUNK, CH)

    degp = _deg_kernel(col, jnp.zeros((NP,), jnp.float32))
    deg0 = degp[0, :N].reshape(N, 1)
    deg1 = degp[1, :N].reshape(N, 1)

    dinv, g1 = _k1(deg0, deg1, x, W_e1)

    s1 = _make_edge_scatter(64)(g1, row, col, jnp.zeros((NP, 64), jnp.float32))
    (g2,) = _k2(s1[0, :N], s1[1, :N], g1, dinv, b_e1.reshape(1, 64), W_e2)

    s2 = _make_edge_scatter(32)(g2, row, col, jnp.zeros((NP, 32), jnp.float32))
    mu, logvar, g3 = _k3(
        s2[0, :N], s2[1, :N], g2, dinv, b_e2.reshape(1, 32),
        W_mu, b_mu.reshape(1, 32), W_lv, b_lv.reshape(1, 32), W_d1,
    )

    s3 = _make_edge_scatter(128)(g3, row, col, jnp.zeros((NP, 128), jnp.float32))
    (g4,) = _k4(s3[0, :N], s3[1, :N], g3, dinv, b_d1.reshape(1, 128), W_d2)

    s4 = _make_edge_scatter(128)(g4, row, col, jnp.zeros((NP, 128), jnp.float32))
    (recon,) = _k5(s4[0, :N], s4[1, :N], g4, dinv, b_d2.reshape(1, 128))

    return recon, mu, logvar


# speedup vs baseline: 21.7239x; 1.0876x over previous
import functools

import jax
import jax.numpy as jnp
from jax import lax
from jax.experimental import pallas as pl
from jax.experimental.pallas import tpu as pltpu
from jax.experimental.pallas import tpu_sc as plsc

N = 10000
NP = 10240
E = 320000
NC = 2
NS = 16
NW = NC * NS
CH = 80
ET = E // NS
NCHUNK = ET // CH
SB = 5
NSB = NCHUNK // SB
EW = E // NW
DNCHUNK = EW // CH
RPT = NP // NS

_MESH = plsc.VectorSubcoreMesh(core_axis_name="c", subcore_axis_name="s")


@functools.cache
def _make_edge_scatter(FH):
    @functools.partial(
        pl.kernel,
        out_type=jax.ShapeDtypeStruct((NC, NP, FH), jnp.float32),
        mesh=_MESH,
        scratch_types=[
            *[pltpu.VMEM((SB, 2, CH), jnp.int32) for _ in range(3)],
            pltpu.VMEM((SB, CH, FH), jnp.float32),
            pltpu.VMEM((SB, CH, FH), jnp.float32),
            pltpu.VMEM_SHARED((NP, FH), jnp.float32),
            *[pltpu.SemaphoreType.DMA for _ in range(7)],
        ],
        compiler_params=pltpu.CompilerParams(use_tc_tiling_on_sc=False),
    )
    def k(g_hbm, idx_hbm, zero_hbm, out_hbm,
          ix0, ix1, ix2, bufa, bufb, acc,
          si0, si1, si2, sga, sgb, ssa, ssb):
        cid = lax.axis_index("c")
        sid = lax.axis_index("s")
        r0 = sid * RPT
        pltpu.sync_copy(zero_hbm.at[pl.ds(r0, RPT)], acc.at[pl.ds(r0, RPT)])

        gh = g_hbm.at[cid]
        ix = (ix0, ix1, ix2)
        si = (si0, si1, si2)
        buf = (bufa, bufb)
        sg = (sga, sgb)
        ss = (ssa, ssb)

        def idx_src(s):
            return idx_hbm.at[sid, pl.ds(s * SB, SB)]

        def issue_idx(s, q):
            pltpu.async_copy(idx_src(s), ix[q], si[q])

        def wait_idx(s, q):
            pltpu.make_async_copy(idx_src(s), ix[q], si[q]).wait()

        def issue_gathers(q, p):
            for b in range(SB):
                pltpu.async_copy(gh.at[ix[q].at[b, 0]], buf[p].at[b], sg[p])

        def wait_gathers(q, p):
            for b in range(SB):
                pltpu.make_async_copy(gh.at[ix[q].at[b, 0]], buf[p].at[b],
                                      sg[p]).wait()

        def issue_scatters(q, p):
            for b in range(SB):
                pltpu.async_copy(buf[p].at[b], acc.at[ix[q].at[b, 1]],
                                 ss[p], add=True)

        def drain_scatters(q, p):
            for b in range(SB):
                pltpu.make_async_copy(buf[p].at[b], acc.at[ix[q].at[b, 1]],
                                      ss[p]).wait()

        plsc.subcore_barrier()
        issue_idx(0, 0)
        issue_idx(1, 1)
        wait_idx(0, 0)
        issue_gathers(0, 0)

        def phase(s, q, p, drain=True, idx2=True, gnext=True):
            wait_gathers(q, p)
            issue_scatters(q, p)
            if drain:
                drain_scatters((q + 2) % 3, 1 - p)
            if idx2:
                issue_idx(s + 2, (q + 2) % 3)
            if gnext:
                wait_idx(s + 1, (q + 1) % 3)
                issue_gathers((q + 1) % 3, 1 - p)

        phase(0, 0, 0, drain=False)

        def body6(k_, carry):
            s0 = 6 * k_ + 1
            phase(s0 + 0, 1, 1)
            phase(s0 + 1, 2, 0)
            phase(s0 + 2, 0, 1)
            phase(s0 + 3, 1, 0)
            phase(s0 + 4, 2, 1)
            phase(s0 + 5, 0, 0)
            return carry

        lax.fori_loop(0, 7, body6, 0)
        for s in range(43, NSB):
            phase(s, s % 3, s % 2, idx2=(s + 2 < NSB), gnext=(s + 1 < NSB))
        drain_scatters((NSB - 1) % 3, (NSB - 1) % 2)
        plsc.subcore_barrier()
        pltpu.sync_copy(acc.at[pl.ds(r0, RPT)], out_hbm.at[cid, pl.ds(r0, RPT)])

    return k


@functools.partial(
    pl.kernel,
    out_type=jax.ShapeDtypeStruct((NC, NP), jnp.float32),
    mesh=_MESH,
    scratch_types=[
        pltpu.VMEM((DNCHUNK, CH), jnp.int32),
        pltpu.VMEM((CH,), jnp.float32),
        pltpu.VMEM_SHARED((NP,), jnp.float32),
    ],
)
def _deg_kernel(col_hbm, zero_hbm, out_hbm, idx_c, ones_v, acc):
    cid = lax.axis_index("c")
    sid = lax.axis_index("s")
    wid = cid * NS + sid
    r0 = sid * RPT
    pltpu.sync_copy(zero_hbm.at[pl.ds(r0, RPT)], acc.at[pl.ds(r0, RPT)])
    pltpu.sync_copy(col_hbm.at[wid], idx_c)
    for i in range(CH // 16):
        ones_v[pl.ds(i * 16, 16)] = jnp.ones((16,), jnp.float32)
    plsc.subcore_barrier()

    def body(j, carry):
        pltpu.sync_copy(ones_v, acc.at[idx_c.at[j]], add=True)
        return carry

    lax.fori_loop(0, DNCHUNK, body, 0)
    plsc.subcore_barrier()
    pltpu.sync_copy(acc.at[pl.ds(r0, RPT)], out_hbm.at[cid, pl.ds(r0, RPT)])


BR = 1000
GRID = (N // BR,)


def _row(f):
    return pl.BlockSpec((BR, f), lambda i: (i, 0))


def _full(a, b):
    return pl.BlockSpec((a, b), lambda i: (0, 0))


def _f32(*shape):
    return jax.ShapeDtypeStruct(shape, jnp.float32)


def _half(fh, c):
    return pl.BlockSpec((1, BR, fh), lambda i: (c, i, 0))


def _row2(f):
    return pl.BlockSpec((BR, f), lambda i, j: (i, 0))


def _full2(a, b):
    return pl.BlockSpec((a, b), lambda i, j: (0, 0))


def _wsel(a, b):
    return pl.BlockSpec((1, a, b), lambda i, j: (j, 0, 0))


def _split_w(w):
    fh = w.shape[1] // 2
    return jnp.stack([w[:, :fh], w[:, fh:]])


def _half2(fh):
    return pl.BlockSpec((1, BR, fh), lambda i, j: (j, i, 0))


def _cat(sa, sb, ga, gb):
    return jnp.concatenate([sa[0] + ga[0], sb[0] + gb[0]], axis=1)


def _k1_body(deg0, deg1, x, w, dinv_o, g_o):
    dinv = lax.rsqrt(deg0[...] + deg1[...] + 1.0)
    dinv_o[...] = dinv
    g_o[0] = dinv * jnp.dot(x[...], w[0], preferred_element_type=jnp.float32)


_k1 = pl.pallas_call(
    _k1_body,
    grid=(N // BR, 2),
    in_specs=[_row2(1), _row2(1), _row2(128), _wsel(128, 32)],
    out_specs=[_row2(1), _half2(32)],
    out_shape=[_f32(N, 1), _f32(2, N, 32)],
)


def _k2_body(sa, sb, ga, gb, dinv, b, w, g2_o):
    dv = dinv[...]
    h = jnp.maximum(dv * _cat(sa, sb, ga, gb) + b[...], 0.0)
    g2_o[0] = dv * jnp.dot(h, w[0], preferred_element_type=jnp.float32)


_k2 = pl.pallas_call(
    _k2_body,
    grid=(N // BR, 2),
    in_specs=[_half2_in := pl.BlockSpec((1, BR, 32), lambda i, j: (0, i, 0)),
              pl.BlockSpec((1, BR, 32), lambda i, j: (1, i, 0)),
              pl.BlockSpec((1, BR, 32), lambda i, j: (0, i, 0)),
              pl.BlockSpec((1, BR, 32), lambda i, j: (1, i, 0)),
              _row2(1), _full2(1, 64), _wsel(64, 16)],
    out_specs=[_half2(16)],
    out_shape=[_f32(2, N, 16)],
)


def _k3_body(sa, sb, ga, gb, dinv, b, wmu, bmu, wlv, blv, wd1,
             mu_o, lv_o, g3_o):
    dv = dinv[...]
    h2 = dv * _cat(sa, sb, ga, gb) + b[...]
    mu = jnp.dot(h2, wmu[...], preferred_element_type=jnp.float32) + bmu[...]
    lv = jnp.dot(h2, wlv[...], preferred_element_type=jnp.float32) + blv[...]
    mu_o[...] = mu
    lv_o[...] = lv
    g3_o[0] = dv * jnp.dot(mu, wd1[0], preferred_element_type=jnp.float32)


_k3 = pl.pallas_call(
    _k3_body,
    grid=(N // BR, 2),
    in_specs=[pl.BlockSpec((1, BR, 16), lambda i, j: (0, i, 0)),
              pl.BlockSpec((1, BR, 16), lambda i, j: (1, i, 0)),
              pl.BlockSpec((1, BR, 16), lambda i, j: (0, i, 0)),
              pl.BlockSpec((1, BR, 16), lambda i, j: (1, i, 0)),
              _row2(1), _full2(1, 32),
              _full2(32, 32), _full2(1, 32), _full2(32, 32), _full2(1, 32),
              _wsel(32, 64)],
    out_specs=[_row2(32), _row2(32), _half2(64)],
    out_shape=[_f32(N, 32), _f32(N, 32), _f32(2, N, 64)],
)


def _k4_body(sa, sb, ga, gb, dinv, b, w, g4_o):
    dv = dinv[...]
    d = jnp.maximum(dv * _cat(sa, sb, ga, gb) + b[...], 0.0)
    g4_o[0] = dv * jnp.dot(d, w[0], preferred_element_type=jnp.float32)


_k4 = pl.pallas_call(
    _k4_body,
    grid=(N // BR, 2),
    in_specs=[pl.BlockSpec((1, BR, 64), lambda i, j: (0, i, 0)),
              pl.BlockSpec((1, BR, 64), lambda i, j: (1, i, 0)),
              pl.BlockSpec((1, BR, 64), lambda i, j: (0, i, 0)),
              pl.BlockSpec((1, BR, 64), lambda i, j: (1, i, 0)),
              _row2(1), _full2(1, 128), _wsel(128, 64)],
    out_specs=[_half2(64)],
    out_shape=[_f32(2, N, 64)],
)


def _k5_body(sa, sb, ga, gb, dinv, b, recon_o):
    recon_o[...] = dinv[...] * _cat(sa, sb, ga, gb) + b[...]


_k5 = pl.pallas_call(
    _k5_body,
    grid=GRID,
    in_specs=[pl.BlockSpec((1, BR, 64), lambda i: (0, i, 0)),
              pl.BlockSpec((1, BR, 64), lambda i: (1, i, 0)),
              pl.BlockSpec((1, BR, 64), lambda i: (0, i, 0)),
              pl.BlockSpec((1, BR, 64), lambda i: (1, i, 0)),
              _row(1), _full(1, 128)],
    out_specs=[_row(128)],
    out_shape=[_f32(N, 128)],
)


def kernel(x, edge_index, W_e1, b_e1, W_e2, b_e2, W_mu, b_mu, W_lv, b_lv,
           W_d1, b_d1, W_d2, b_d2):
    ei = edge_index.astype(jnp.int32)
    idxcat = jnp.stack(
        [ei[0].reshape(NS, NCHUNK, CH), ei[1].reshape(NS, NCHUNK, CH)], axis=2)
    colw = ei[1].reshape(NW, DNCHUNK, CH)

    degp = _deg_kernel(colw, jnp.zeros((NP,), jnp.float32))
    deg0 = degp[0, :N].reshape(N, 1)
    deg1 = degp[1, :N].reshape(N, 1)

    dinv, g1 = _k1(deg0, deg1, x, _split_w(W_e1))

    s1 = _make_edge_scatter(32)(g1, idxcat, jnp.zeros((NP, 32), jnp.float32))
    s1 = s1[:, :N]
    (g2,) = _k2(s1, s1, g1, g1, dinv, b_e1.reshape(1, 64), _split_w(W_e2))

    s2 = _make_edge_scatter(16)(g2, idxcat, jnp.zeros((NP, 16), jnp.float32))
    s2 = s2[:, :N]
    mu, logvar, g3 = _k3(
        s2, s2, g2, g2, dinv, b_e2.reshape(1, 32),
        W_mu, b_mu.reshape(1, 32), W_lv, b_lv.reshape(1, 32), _split_w(W_d1),
    )

    s3 = _make_edge_scatter(64)(g3, idxcat, jnp.zeros((NP, 64), jnp.float32))
    s3 = s3[:, :N]
    (g4,) = _k4(s3, s3, g3, g3, dinv, b_d1.reshape(1, 128), _split_w(W_d2))

    s4 = _make_edge_scatter(64)(g4, idxcat, jnp.zeros((NP, 64), jnp.float32))
    s4 = s4[:, :N]
    (recon,) = _k5(s4, s4, g4, g4, dinv, b_d2.reshape(1, 128))

    return recon, mu, logvar


# speedup vs baseline: 24.8662x; 1.1446x over previous
import functools

import jax
import jax.numpy as jnp
from jax import lax
from jax.experimental import pallas as pl
from jax.experimental.pallas import tpu as pltpu
from jax.experimental.pallas import tpu_sc as plsc

N = 10000
NP = 10240
E = 320000
NC = 2
NS = 16
NW = NC * NS
CH = 80
ET = E // NS
NCHUNK = ET // CH
SB = 5
NSB = NCHUNK // SB
DNCHUNK = NCHUNK // NC
RPT = NP // NS
RPN = N // NS

_MESH = plsc.VectorSubcoreMesh(core_axis_name="c", subcore_axis_name="s")


@functools.cache
def _make_edge_scatter(FH):
    @functools.partial(
        pl.kernel,
        out_type=jax.ShapeDtypeStruct((N, 2 * FH), jnp.float32),
        mesh=_MESH,
        scratch_types=[
            *[pltpu.VMEM((SB, 2, CH), jnp.int32) for _ in range(3)],
            pltpu.VMEM((SB, CH, FH), jnp.float32),
            pltpu.VMEM((SB, CH, FH), jnp.float32),
            pltpu.VMEM_SHARED((NP, FH), jnp.float32),
            *[pltpu.SemaphoreType.DMA for _ in range(7)],
        ],
        compiler_params=pltpu.CompilerParams(use_tc_tiling_on_sc=False),
    )
    def k(g_hbm, idx_hbm, zero_hbm, out_hbm,
          ix0, ix1, ix2, bufa, bufb, acc,
          si0, si1, si2, sga, sgb, ssa, ssb):
        cid = lax.axis_index("c")
        sid = lax.axis_index("s")
        r0 = sid * RPT
        pltpu.sync_copy(zero_hbm.at[pl.ds(r0, RPT)], acc.at[pl.ds(r0, RPT)])

        gh = g_hbm.at[cid]
        ix = (ix0, ix1, ix2)
        si = (si0, si1, si2)
        buf = (bufa, bufb)
        sg = (sga, sgb)
        ss = (ssa, ssb)

        def idx_src(s):
            return idx_hbm.at[sid, pl.ds(s * SB, SB)]

        def issue_idx(s, q):
            pltpu.async_copy(idx_src(s), ix[q], si[q])

        def wait_idx(s, q):
            pltpu.make_async_copy(idx_src(s), ix[q], si[q]).wait()

        def issue_gathers(q, p):
            for b in range(SB):
                pltpu.async_copy(gh.at[ix[q].at[b, 0]], buf[p].at[b], sg[p])

        def wait_gathers(q, p):
            for b in range(SB):
                pltpu.make_async_copy(gh.at[ix[q].at[b, 0]], buf[p].at[b],
                                      sg[p]).wait()

        def issue_scatters(q, p):
            for b in range(SB):
                pltpu.async_copy(buf[p].at[b], acc.at[ix[q].at[b, 1]],
                                 ss[p], add=True)

        def drain_scatters(q, p):
            for b in range(SB):
                pltpu.make_async_copy(buf[p].at[b], acc.at[ix[q].at[b, 1]],
                                      ss[p]).wait()

        plsc.subcore_barrier()
        issue_idx(0, 0)
        issue_idx(1, 1)
        wait_idx(0, 0)
        issue_gathers(0, 0)

        def phase(s, q, p, drain=True, idx2=True, gnext=True):
            wait_gathers(q, p)
            issue_scatters(q, p)
            if drain:
                drain_scatters((q + 2) % 3, 1 - p)
            if idx2:
                issue_idx(s + 2, (q + 2) % 3)
            if gnext:
                wait_idx(s + 1, (q + 1) % 3)
                issue_gathers((q + 1) % 3, 1 - p)

        phase(0, 0, 0, drain=False)

        def body6(k_, carry):
            s0 = 6 * k_ + 1
            phase(s0 + 0, 1, 1)
            phase(s0 + 1, 2, 0)
            phase(s0 + 2, 0, 1)
            phase(s0 + 3, 1, 0)
            phase(s0 + 4, 2, 1)
            phase(s0 + 5, 0, 0)
            return carry

        lax.fori_loop(0, 7, body6, 0)
        for s in range(43, NSB):
            phase(s, s % 3, s % 2, idx2=(s + 2 < NSB), gnext=(s + 1 < NSB))
        drain_scatters((NSB - 1) % 3, (NSB - 1) % 2)
        plsc.subcore_barrier()
        r1 = sid * RPN
        pltpu.sync_copy(acc.at[pl.ds(r1, RPN)],
                        out_hbm.at[pl.ds(r1, RPN), pl.ds(cid * FH, FH)])

    return k


@functools.partial(
    pl.kernel,
    out_type=jax.ShapeDtypeStruct((NC, NP), jnp.float32),
    mesh=_MESH,
    scratch_types=[
        pltpu.VMEM((DNCHUNK, CH), jnp.int32),
        pltpu.VMEM((CH,), jnp.float32),
        pltpu.VMEM_SHARED((NP,), jnp.float32),
    ],
)
def _deg_kernel(idx_hbm, zero_hbm, out_hbm, idx_c, ones_v, acc):
    cid = lax.axis_index("c")
    sid = lax.axis_index("s")
    r0 = sid * RPT
    pltpu.sync_copy(zero_hbm.at[pl.ds(r0, RPT)], acc.at[pl.ds(r0, RPT)])
    pltpu.sync_copy(idx_hbm.at[sid, pl.ds(cid * DNCHUNK, DNCHUNK), 1], idx_c)
    for i in range(CH // 16):
        ones_v[pl.ds(i * 16, 16)] = jnp.ones((16,), jnp.float32)
    plsc.subcore_barrier()

    def body(j, carry):
        pltpu.sync_copy(ones_v, acc.at[idx_c.at[j]], add=True)
        return carry

    lax.fori_loop(0, DNCHUNK, body, 0)
    plsc.subcore_barrier()
    pltpu.sync_copy(acc.at[pl.ds(r0, RPT)], out_hbm.at[cid, pl.ds(r0, RPT)])


BR = 2000
GRID = (N // BR,)


def _row(f):
    return pl.BlockSpec((BR, f), lambda i: (i, 0))


def _full(a, b):
    return pl.BlockSpec((a, b), lambda i: (0, 0))


def _f32(*shape):
    return jax.ShapeDtypeStruct(shape, jnp.float32)


def _k1_body(deg0, deg1, x, w, dinv_o, g_o):
    dinv = lax.rsqrt(deg0[...] + deg1[...] + 1.0)
    dinv_o[...] = dinv
    g_o[...] = dinv * jnp.dot(x[...], w[...], preferred_element_type=jnp.float32)


_k1 = pl.pallas_call(
    _k1_body,
    grid=GRID,
    in_specs=[_row(1), _row(1), _row(128), _full(128, 64)],
    out_specs=[_row(1), _row(64)],
    out_shape=[_f32(N, 1), _f32(N, 64)],
)


def _k2_body(s, g, dinv, b, w, g2_o):
    dv = dinv[...]
    h = jnp.maximum(dv * (s[...] + g[...]) + b[...], 0.0)
    g2_o[...] = dv * jnp.dot(h, w[...], preferred_element_type=jnp.float32)


_k2 = pl.pallas_call(
    _k2_body,
    grid=GRID,
    in_specs=[_row(64), _row(64), _row(1), _full(1, 64), _full(64, 32)],
    out_specs=[_row(32)],
    out_shape=[_f32(N, 32)],
)


def _k3_body(s, g, dinv, b, wmu, bmu, wlv, blv, wd1, mu_o, lv_o, g3_o):
    dv = dinv[...]
    h2 = dv * (s[...] + g[...]) + b[...]
    mu = jnp.dot(h2, wmu[...], preferred_element_type=jnp.float32) + bmu[...]
    lv = jnp.dot(h2, wlv[...], preferred_element_type=jnp.float32) + blv[...]
    mu_o[...] = mu
    lv_o[...] = lv
    g3_o[...] = dv * jnp.dot(mu, wd1[...], preferred_element_type=jnp.float32)


_k3 = pl.pallas_call(
    _k3_body,
    grid=GRID,
    in_specs=[
        _row(32), _row(32), _row(1), _full(1, 32),
        _full(32, 32), _full(1, 32), _full(32, 32), _full(1, 32), _full(32, 128),
    ],
    out_specs=[_row(32), _row(32), _row(128)],
    out_shape=[_f32(N, 32), _f32(N, 32), _f32(N, 128)],
)


def _k4_body(s, g, dinv, b, w, g4_o):
    dv = dinv[...]
    d = jnp.maximum(dv * (s[...] + g[...]) + b[...], 0.0)
    g4_o[...] = dv * jnp.dot(d, w[...], preferred_element_type=jnp.float32)


_k4 = pl.pallas_call(
    _k4_body,
    grid=GRID,
    in_specs=[_row(128), _row(128), _row(1), _full(1, 128), _full(128, 128)],
    out_specs=[_row(128)],
    out_shape=[_f32(N, 128)],
)


def _k5_body(s, g, dinv, b, recon_o):
    recon_o[...] = dinv[...] * (s[...] + g[...]) + b[...]


_k5 = pl.pallas_call(
    _k5_body,
    grid=GRID,
    in_specs=[_row(128), _row(128), _row(1), _full(1, 128)],
    out_specs=[_row(128)],
    out_shape=[_f32(N, 128)],
)


def _halves(g):
    fh = g.shape[1] // 2
    return jnp.stack([g[:, :fh], g[:, fh:]])


def kernel(x, edge_index, W_e1, b_e1, W_e2, b_e2, W_mu, b_mu, W_lv, b_lv,
           W_d1, b_d1, W_d2, b_d2):
    ei = edge_index.astype(jnp.int32)
    idxcat = jnp.stack(
        [ei[0].reshape(NS, NCHUNK, CH), ei[1].reshape(NS, NCHUNK, CH)], axis=2)

    degp = _deg_kernel(idxcat, jnp.zeros((NP,), jnp.float32))
    deg0 = degp[0, :N].reshape(N, 1)
    deg1 = degp[1, :N].reshape(N, 1)

    dinv, g1 = _k1(deg0, deg1, x, W_e1)

    s1 = _make_edge_scatter(32)(_halves(g1), idxcat,
                                jnp.zeros((NP, 32), jnp.float32))
    (g2,) = _k2(s1, g1, dinv, b_e1.reshape(1, 64), W_e2)

    s2 = _make_edge_scatter(16)(_halves(g2), idxcat,
                                jnp.zeros((NP, 16), jnp.float32))
    mu, logvar, g3 = _k3(
        s2, g2, dinv, b_e2.reshape(1, 32),
        W_mu, b_mu.reshape(1, 32), W_lv, b_lv.reshape(1, 32), W_d1,
    )

    s3 = _make_edge_scatter(64)(_halves(g3), idxcat,
                                jnp.zeros((NP, 64), jnp.float32))
    (g4,) = _k4(s3, g3, dinv, b_d1.reshape(1, 128), W_d2)

    s4 = _make_edge_scatter(64)(_halves(g4), idxcat,
                                jnp.zeros((NP, 64), jnp.float32))
    (recon,) = _k5(s4, g4, dinv, b_d2.reshape(1, 128))

    return recon, mu, logvar


# speedup vs baseline: 26.5442x; 1.0675x over previous
import functools

import jax
import jax.numpy as jnp
from jax import lax
from jax.experimental import pallas as pl
from jax.experimental.pallas import tpu as pltpu
from jax.experimental.pallas import tpu_sc as plsc

N = 10000
NP = 10240
E = 320000
NC = 2
NS = 16
NW = NC * NS
CH = 80
ET = E // NS
NCHUNK = ET // CH
SB = 5
NSB = NCHUNK // SB
DNCHUNK = NCHUNK // NC
RPT = NP // NS
RPN = N // NS

_MESH = plsc.VectorSubcoreMesh(core_axis_name="c", subcore_axis_name="s")


@functools.cache
def _make_edge_scatter(FH):
    @functools.partial(
        pl.kernel,
        out_type=jax.ShapeDtypeStruct((N, 2 * FH), jnp.float32),
        mesh=_MESH,
        scratch_types=[
            *[pltpu.VMEM((SB, 2, CH), jnp.int32) for _ in range(3)],
            pltpu.VMEM((SB, CH, FH), jnp.float32),
            pltpu.VMEM((SB, CH, FH), jnp.float32),
            pltpu.VMEM_SHARED((NP, FH), jnp.float32),
            *[pltpu.SemaphoreType.DMA for _ in range(7)],
        ],
        compiler_params=pltpu.CompilerParams(use_tc_tiling_on_sc=False),
    )
    def k(g_hbm, idx_hbm, zero_hbm, out_hbm,
          ix0, ix1, ix2, bufa, bufb, acc,
          si0, si1, si2, sga, sgb, ssa, ssb):
        cid = lax.axis_index("c")
        sid = lax.axis_index("s")
        r0 = sid * RPT
        pltpu.sync_copy(zero_hbm.at[pl.ds(r0, RPT)], acc.at[pl.ds(r0, RPT)])

        ix = (ix0, ix1, ix2)
        si = (si0, si1, si2)
        buf = (bufa, bufb)
        sg = (sga, sgb)
        ss = (ssa, ssb)

        def idx_src(s):
            return idx_hbm.at[cid, sid, pl.ds(s * SB, SB)]

        def issue_idx(s, q):
            pltpu.async_copy(idx_src(s), ix[q], si[q])

        def wait_idx(s, q):
            pltpu.make_async_copy(idx_src(s), ix[q], si[q]).wait()

        def issue_gathers(q, p):
            for b in range(SB):
                pltpu.async_copy(g_hbm.at[ix[q].at[b, 0]], buf[p].at[b], sg[p])

        def wait_gathers(q, p):
            for b in range(SB):
                pltpu.make_async_copy(g_hbm.at[ix[q].at[b, 0]], buf[p].at[b],
                                      sg[p]).wait()

        def issue_scatters(q, p):
            for b in range(SB):
                pltpu.async_copy(buf[p].at[b], acc.at[ix[q].at[b, 1]],
                                 ss[p], add=True)

        def drain_scatters(q, p):
            for b in range(SB):
                pltpu.make_async_copy(buf[p].at[b], acc.at[ix[q].at[b, 1]],
                                      ss[p]).wait()

        plsc.subcore_barrier()
        issue_idx(0, 0)
        issue_idx(1, 1)
        wait_idx(0, 0)
        issue_gathers(0, 0)

        def phase(s, q, p, drain=True, idx2=True, gnext=True):
            wait_gathers(q, p)
            issue_scatters(q, p)
            if drain:
                drain_scatters((q + 2) % 3, 1 - p)
            if idx2:
                issue_idx(s + 2, (q + 2) % 3)
            if gnext:
                wait_idx(s + 1, (q + 1) % 3)
                issue_gathers((q + 1) % 3, 1 - p)

        phase(0, 0, 0, drain=False)

        def body6(k_, carry):
            s0 = 6 * k_ + 1
            phase(s0 + 0, 1, 1)
            phase(s0 + 1, 2, 0)
            phase(s0 + 2, 0, 1)
            phase(s0 + 3, 1, 0)
            phase(s0 + 4, 2, 1)
            phase(s0 + 5, 0, 0)
            return carry

        lax.fori_loop(0, 7, body6, 0)
        for s in range(43, NSB):
            phase(s, s % 3, s % 2, idx2=(s + 2 < NSB), gnext=(s + 1 < NSB))
        drain_scatters((NSB - 1) % 3, (NSB - 1) % 2)
        plsc.subcore_barrier()
        r1 = sid * RPN
        pltpu.sync_copy(acc.at[pl.ds(r1, RPN)],
                        out_hbm.at[pl.ds(r1, RPN), pl.ds(cid * FH, FH)])

    return k


@functools.partial(
    pl.kernel,
    out_type=jax.ShapeDtypeStruct((NC, NP), jnp.float32),
    mesh=_MESH,
    scratch_types=[
        pltpu.VMEM((DNCHUNK, 2, CH), jnp.int32),
        pltpu.VMEM((CH,), jnp.float32),
        pltpu.VMEM_SHARED((NP,), jnp.float32),
    ],
)
def _deg_kernel(idx_hbm, zero_hbm, out_hbm, idx_c, ones_v, acc):
    cid = lax.axis_index("c")
    sid = lax.axis_index("s")
    r0 = sid * RPT
    pltpu.sync_copy(zero_hbm.at[pl.ds(r0, RPT)], acc.at[pl.ds(r0, RPT)])
    pltpu.sync_copy(idx_hbm.at[cid, sid, pl.ds(cid * DNCHUNK, DNCHUNK)], idx_c)
    for i in range(CH // 16):
        ones_v[pl.ds(i * 16, 16)] = jnp.ones((16,), jnp.float32)
    plsc.subcore_barrier()

    def body(j, carry):
        pltpu.sync_copy(ones_v, acc.at[idx_c.at[j, 1]], add=True)
        return carry

    lax.fori_loop(0, DNCHUNK, body, 0)
    plsc.subcore_barrier()
    pltpu.sync_copy(acc.at[pl.ds(r0, RPT)], out_hbm.at[cid, pl.ds(r0, RPT)])


BR = 2000
GRID = (N // BR,)


def _row(f):
    return pl.BlockSpec((BR, f), lambda i: (i, 0))


def _full(a, b):
    return pl.BlockSpec((a, b), lambda i: (0, 0))


def _f32(*shape):
    return jax.ShapeDtypeStruct(shape, jnp.float32)


def _k1_body(deg0, deg1, x, w, dinv_o, g_o):
    dinv = lax.rsqrt(deg0[...] + deg1[...] + 1.0)
    dinv_o[...] = dinv
    g_o[...] = dinv * jnp.dot(x[...], w[...], preferred_element_type=jnp.float32)


_k1 = pl.pallas_call(
    _k1_body,
    grid=GRID,
    in_specs=[_row(1), _row(1), _row(128), _full(128, 64)],
    out_specs=[_row(1), _row(64)],
    out_shape=[_f32(N, 1), _f32(N, 64)],
)


def _k2_body(s, g, dinv, b, w, g2_o):
    dv = dinv[...]
    h = jnp.maximum(dv * (s[...] + g[...]) + b[...], 0.0)
    g2_o[...] = dv * jnp.dot(h, w[...], preferred_element_type=jnp.float32)


_k2 = pl.pallas_call(
    _k2_body,
    grid=GRID,
    in_specs=[_row(64), _row(64), _row(1), _full(1, 64), _full(64, 32)],
    out_specs=[_row(32)],
    out_shape=[_f32(N, 32)],
)


def _k3_body(s, g, dinv, b, wmu, bmu, wlv, blv, wd1, mu_o, lv_o, g3_o):
    dv = dinv[...]
    h2 = dv * (s[...] + g[...]) + b[...]
    mu = jnp.dot(h2, wmu[...], preferred_element_type=jnp.float32) + bmu[...]
    lv = jnp.dot(h2, wlv[...], preferred_element_type=jnp.float32) + blv[...]
    mu_o[...] = mu
    lv_o[...] = lv
    g3_o[...] = dv * jnp.dot(mu, wd1[...], preferred_element_type=jnp.float32)


_k3 = pl.pallas_call(
    _k3_body,
    grid=GRID,
    in_specs=[
        _row(32), _row(32), _row(1), _full(1, 32),
        _full(32, 32), _full(1, 32), _full(32, 32), _full(1, 32), _full(32, 128),
    ],
    out_specs=[_row(32), _row(32), _row(128)],
    out_shape=[_f32(N, 32), _f32(N, 32), _f32(N, 128)],
)


def _k4_body(s, g, dinv, b, w, g4_o):
    dv = dinv[...]
    d = jnp.maximum(dv * (s[...] + g[...]) + b[...], 0.0)
    g4_o[...] = dv * jnp.dot(d, w[...], preferred_element_type=jnp.float32)


_k4 = pl.pallas_call(
    _k4_body,
    grid=GRID,
    in_specs=[_row(128), _row(128), _row(1), _full(1, 128), _full(128, 128)],
    out_specs=[_row(128)],
    out_shape=[_f32(N, 128)],
)


def _k5_body(s, g, dinv, b, recon_o):
    recon_o[...] = dinv[...] * (s[...] + g[...]) + b[...]


_k5 = pl.pallas_call(
    _k5_body,
    grid=GRID,
    in_specs=[_row(128), _row(128), _row(1), _full(1, 128)],
    out_specs=[_row(128)],
    out_shape=[_f32(N, 128)],
)


def _twin(g):
    return g.reshape(2 * N, g.shape[1] // 2)


def kernel(x, edge_index, W_e1, b_e1, W_e2, b_e2, W_mu, b_mu, W_lv, b_lv,
           W_d1, b_d1, W_d2, b_d2):
    ei = edge_index.astype(jnp.int32)
    row = ei[0].reshape(NS, NCHUNK, CH)
    col = ei[1].reshape(NS, NCHUNK, CH)
    idx2 = jnp.stack([jnp.stack([2 * row, col], axis=2),
                      jnp.stack([2 * row + 1, col], axis=2)])

    degp = _deg_kernel(idx2, jnp.zeros((NP,), jnp.float32))
    deg0 = degp[0, :N].reshape(N, 1)
    deg1 = degp[1, :N].reshape(N, 1)

    dinv, g1 = _k1(deg0, deg1, x, W_e1)

    s1 = _make_edge_scatter(32)(_twin(g1), idx2,
                                jnp.zeros((NP, 32), jnp.float32))
    (g2,) = _k2(s1, g1, dinv, b_e1.reshape(1, 64), W_e2)

    s2 = _make_edge_scatter(16)(_twin(g2), idx2,
                                jnp.zeros((NP, 16), jnp.float32))
    mu, logvar, g3 = _k3(
        s2, g2, dinv, b_e2.reshape(1, 32),
        W_mu, b_mu.reshape(1, 32), W_lv, b_lv.reshape(1, 32), W_d1,
    )

    s3 = _make_edge_scatter(64)(_twin(g3), idx2,
                                jnp.zeros((NP, 64), jnp.float32))
    (g4,) = _k4(s3, g3, dinv, b_d1.reshape(1, 128), W_d2)

    s4 = _make_edge_scatter(64)(_twin(g4), idx2,
                                jnp.zeros((NP, 64), jnp.float32))
    (recon,) = _k5(s4, g4, dinv, b_d2.reshape(1, 128))

    return recon, mu, logvar


# speedup vs baseline: 28.4965x; 1.0735x over previous
import functools

import jax
import jax.numpy as jnp
from jax import lax
from jax.experimental import pallas as pl
from jax.experimental.pallas import tpu as pltpu
from jax.experimental.pallas import tpu_sc as plsc

N = 10000
NP = 10240
E = 320000
NC = 2
NS = 16
NW = NC * NS
CH = 80
ET = E // NS
NCHUNK = ET // CH
SB = 5
NSB = NCHUNK // SB
DNCHUNK = NCHUNK // NC
RPT = NP // NS
RPN = N // NS

_MESH = plsc.VectorSubcoreMesh(core_axis_name="c", subcore_axis_name="s")


@functools.cache
def _make_edge_scatter(FH):
    @functools.partial(
        pl.kernel,
        out_type=jax.ShapeDtypeStruct((N, 2 * FH), jnp.float32),
        mesh=_MESH,
        scratch_types=[
            *[pltpu.VMEM((SB, CH), jnp.int32) for _ in range(3)],
            *[pltpu.VMEM((SB, CH), jnp.int32) for _ in range(3)],
            pltpu.VMEM((SB, CH, FH), jnp.float32),
            pltpu.VMEM((SB, CH, FH), jnp.float32),
            pltpu.VMEM_SHARED((NP, FH), jnp.float32),
            *[pltpu.SemaphoreType.DMA for _ in range(7)],
        ],
        compiler_params=pltpu.CompilerParams(use_tc_tiling_on_sc=False),
    )
    def k(g_hbm, row_hbm, col_hbm, zero_hbm, out_hbm,
          ixr0, ixr1, ixr2, ixc0, ixc1, ixc2, bufa, bufb, acc,
          si0, si1, si2, sga, sgb, ssa, ssb):
        cid = lax.axis_index("c")
        sid = lax.axis_index("s")
        r0 = sid * RPT
        pltpu.sync_copy(zero_hbm.at[pl.ds(r0, RPT)], acc.at[pl.ds(r0, RPT)])

        ixr = (ixr0, ixr1, ixr2)
        ixc = (ixc0, ixc1, ixc2)
        si = (si0, si1, si2)
        buf = (bufa, bufb)
        sg = (sga, sgb)
        ss = (ssa, ssb)

        def issue_idx(s, q):
            pltpu.async_copy(row_hbm.at[cid, sid, s], ixr[q], si[q])
            pltpu.async_copy(col_hbm.at[sid, s], ixc[q], si[q])

        def wait_idx(s, q):
            pltpu.make_async_copy(row_hbm.at[cid, sid, s], ixr[q], si[q]).wait()
            pltpu.make_async_copy(col_hbm.at[sid, s], ixc[q], si[q]).wait()

        def issue_gathers(q, p):
            for b in range(SB):
                pltpu.async_copy(g_hbm.at[ixr[q].at[b]], buf[p].at[b], sg[p])

        def wait_gathers(q, p):
            for b in range(SB):
                pltpu.make_async_copy(g_hbm.at[ixr[q].at[b]], buf[p].at[b],
                                      sg[p]).wait()

        def issue_scatters(q, p):
            for b in range(SB):
                pltpu.async_copy(buf[p].at[b], acc.at[ixc[q].at[b]],
                                 ss[p], add=True)

        def drain_scatters(q, p):
            for b in range(SB):
                pltpu.make_async_copy(buf[p].at[b], acc.at[ixc[q].at[b]],
                                      ss[p]).wait()

        plsc.subcore_barrier()
        issue_idx(0, 0)
        issue_idx(1, 1)
        wait_idx(0, 0)
        issue_gathers(0, 0)

        def phase(s, q, p, drain=True, idx2=True, gnext=True):
            wait_gathers(q, p)
            issue_scatters(q, p)
            if drain:
                drain_scatters((q + 2) % 3, 1 - p)
            if idx2:
                issue_idx(s + 2, (q + 2) % 3)
            if gnext:
                wait_idx(s + 1, (q + 1) % 3)
                issue_gathers((q + 1) % 3, 1 - p)

        phase(0, 0, 0, drain=False)

        def body6(k_, carry):
            s0 = 6 * k_ + 1
            phase(s0 + 0, 1, 1)
            phase(s0 + 1, 2, 0)
            phase(s0 + 2, 0, 1)
            phase(s0 + 3, 1, 0)
            phase(s0 + 4, 2, 1)
            phase(s0 + 5, 0, 0)
            return carry

        lax.fori_loop(0, 7, body6, 0)
        for s in range(43, NSB):
            phase(s, s % 3, s % 2, idx2=(s + 2 < NSB), gnext=(s + 1 < NSB))
        drain_scatters((NSB - 1) % 3, (NSB - 1) % 2)
        plsc.subcore_barrier()
        r1 = sid * RPN
        pltpu.sync_copy(acc.at[pl.ds(r1, RPN)],
                        out_hbm.at[pl.ds(r1, RPN), pl.ds(cid * FH, FH)])

    return k


@functools.partial(
    pl.kernel,
    out_type=jax.ShapeDtypeStruct((NC, NP), jnp.float32),
    mesh=_MESH,
    scratch_types=[
        pltpu.VMEM((NSB // NC, SB, CH), jnp.int32),
        pltpu.VMEM((CH,), jnp.float32),
        pltpu.VMEM_SHARED((NP,), jnp.float32),
    ],
)
def _deg_kernel(col_hbm, zero_hbm, out_hbm, idx_c, ones_v, acc):
    cid = lax.axis_index("c")
    sid = lax.axis_index("s")
    r0 = sid * RPT
    pltpu.sync_copy(zero_hbm.at[pl.ds(r0, RPT)], acc.at[pl.ds(r0, RPT)])
    pltpu.sync_copy(col_hbm.at[sid, pl.ds(cid * (NSB // NC), NSB // NC)], idx_c)
    for i in range(CH // 16):
        ones_v[pl.ds(i * 16, 16)] = jnp.ones((16,), jnp.float32)
    plsc.subcore_barrier()

    def body(j, carry):
        for b in range(SB):
            pltpu.sync_copy(ones_v, acc.at[idx_c.at[j, b]], add=True)
        return carry

    lax.fori_loop(0, NSB // NC, body, 0)
    plsc.subcore_barrier()
    pltpu.sync_copy(acc.at[pl.ds(r0, RPT)], out_hbm.at[cid, pl.ds(r0, RPT)])


BR = 2000
GRID = (N // BR,)


def _row(f):
    return pl.BlockSpec((BR, f), lambda i: (i, 0))


def _full(a, b):
    return pl.BlockSpec((a, b), lambda i: (0, 0))


def _f32(*shape):
    return jax.ShapeDtypeStruct(shape, jnp.float32)


def _k1_body(deg0, deg1, x, w, dinv_o, g_o):
    dinv = lax.rsqrt(deg0[...] + deg1[...] + 1.0)
    dinv_o[...] = dinv
    g_o[...] = dinv * jnp.dot(x[...], w[...], preferred_element_type=jnp.float32)


_k1 = pl.pallas_call(
    _k1_body,
    grid=GRID,
    in_specs=[_row(1), _row(1), _row(128), _full(128, 64)],
    out_specs=[_row(1), _row(64)],
    out_shape=[_f32(N, 1), _f32(N, 64)],
)


def _k2_body(s, g, dinv, b, w, g2_o):
    dv = dinv[...]
    h = jnp.maximum(dv * (s[...] + g[...]) + b[...], 0.0)
    g2_o[...] = dv * jnp.dot(h, w[...], preferred_element_type=jnp.float32)


_k2 = pl.pallas_call(
    _k2_body,
    grid=GRID,
    in_specs=[_row(64), _row(64), _row(1), _full(1, 64), _full(64, 32)],
    out_specs=[_row(32)],
    out_shape=[_f32(N, 32)],
)


def _k3_body(s, g, dinv, b, wmu, bmu, wlv, blv, wd1, mu_o, lv_o, g3_o):
    dv = dinv[...]
    h2 = dv * (s[...] + g[...]) + b[...]
    mu = jnp.dot(h2, wmu[...], preferred_element_type=jnp.float32) + bmu[...]
    lv = jnp.dot(h2, wlv[...], preferred_element_type=jnp.float32) + blv[...]
    mu_o[...] = mu
    lv_o[...] = lv
    g3_o[...] = dv * jnp.dot(mu, wd1[...], preferred_element_type=jnp.float32)


_k3 = pl.pallas_call(
    _k3_body,
    grid=GRID,
    in_specs=[
        _row(32), _row(32), _row(1), _full(1, 32),
        _full(32, 32), _full(1, 32), _full(32, 32), _full(1, 32), _full(32, 128),
    ],
    out_specs=[_row(32), _row(32), _row(128)],
    out_shape=[_f32(N, 32), _f32(N, 32), _f32(N, 128)],
)


def _k4_body(s, g, dinv, b, w, g4_o):
    dv = dinv[...]
    d = jnp.maximum(dv * (s[...] + g[...]) + b[...], 0.0)
    g4_o[...] = dv * jnp.dot(d, w[...], preferred_element_type=jnp.float32)


_k4 = pl.pallas_call(
    _k4_body,
    grid=GRID,
    in_specs=[_row(128), _row(128), _row(1), _full(1, 128), _full(128, 128)],
    out_specs=[_row(128)],
    out_shape=[_f32(N, 128)],
)


def _k5_body(s, g, dinv, b, recon_o):
    recon_o[...] = dinv[...] * (s[...] + g[...]) + b[...]


_k5 = pl.pallas_call(
    _k5_body,
    grid=GRID,
    in_specs=[_row(128), _row(128), _row(1), _full(1, 128)],
    out_specs=[_row(128)],
    out_shape=[_f32(N, 128)],
)


def _twin(g):
    return g.reshape(2 * N, g.shape[1] // 2)


def kernel(x, edge_index, W_e1, b_e1, W_e2, b_e2, W_mu, b_mu, W_lv, b_lv,
           W_d1, b_d1, W_d2, b_d2):
    ei = edge_index.astype(jnp.int32)
    row = ei[0].reshape(NS, NSB, SB, CH)
    rowx = jnp.stack([2 * row, 2 * row + 1])
    colx = ei[1].reshape(NS, NSB, SB, CH)

    degp = _deg_kernel(colx, jnp.zeros((NP,), jnp.float32))
    deg0 = degp[0, :N].reshape(N, 1)
    deg1 = degp[1, :N].reshape(N, 1)

    dinv, g1 = _k1(deg0, deg1, x, W_e1)

    s1 = _make_edge_scatter(32)(_twin(g1), rowx, colx,
                                jnp.zeros((NP, 32), jnp.float32))
    (g2,) = _k2(s1, g1, dinv, b_e1.reshape(1, 64), W_e2)

    s2 = _make_edge_scatter(16)(_twin(g2), rowx, colx,
                                jnp.zeros((NP, 16), jnp.float32))
    mu, logvar, g3 = _k3(
        s2, g2, dinv, b_e2.reshape(1, 32),
        W_mu, b_mu.reshape(1, 32), W_lv, b_lv.reshape(1, 32), W_d1,
    )

    s3 = _make_edge_scatter(64)(_twin(g3), rowx, colx,
                                jnp.zeros((NP, 64), jnp.float32))
    (g4,) = _k4(s3, g3, dinv, b_d1.reshape(1, 128), W_d2)

    s4 = _make_edge_scatter(64)(_twin(g4), rowx, colx,
                                jnp.zeros((NP, 64), jnp.float32))
    (recon,) = _k5(s4, g4, dinv, b_d2.reshape(1, 128))

    return recon, mu, logvar


# speedup vs baseline: 29.8702x; 1.0482x over previous
import functools

import jax
import jax.numpy as jnp
from jax import lax
from jax.experimental import pallas as pl
from jax.experimental.pallas import tpu as pltpu
from jax.experimental.pallas import tpu_sc as plsc

N = 10000
NP = 10240
E = 320000
NC = 2
NS = 16
NW = NC * NS
CH = 80
ET = E // NS
NCHUNK = ET // CH
SB = 5
NSB = NCHUNK // SB
DNCHUNK = NCHUNK // NC
RPT = NP // NS
RPN = N // NS

_MESH = plsc.VectorSubcoreMesh(core_axis_name="c", subcore_axis_name="s")


@functools.cache
def _make_edge_scatter(FH):
    SBF = 10 if FH <= 32 else SB
    NSBF = NCHUNK // SBF
    N6 = (NSBF - 3) // 6

    @functools.partial(
        pl.kernel,
        out_type=jax.ShapeDtypeStruct((N, 2 * FH), jnp.float32),
        mesh=_MESH,
        scratch_types=[
            *[pltpu.VMEM((SBF, CH), jnp.int32) for _ in range(3)],
            *[pltpu.VMEM((SBF, CH), jnp.int32) for _ in range(3)],
            pltpu.VMEM((SBF, CH, FH), jnp.float32),
            pltpu.VMEM((SBF, CH, FH), jnp.float32),
            pltpu.VMEM_SHARED((NP, FH), jnp.float32),
            *[pltpu.SemaphoreType.DMA for _ in range(7)],
        ],
        compiler_params=pltpu.CompilerParams(use_tc_tiling_on_sc=False),
    )
    def k(g_hbm, row_hbm, col_hbm, zero_hbm, out_hbm,
          ixr0, ixr1, ixr2, ixc0, ixc1, ixc2, bufa, bufb, acc,
          si0, si1, si2, sga, sgb, ssa, ssb):
        cid = lax.axis_index("c")
        sid = lax.axis_index("s")
        r0 = sid * RPT
        pltpu.sync_copy(zero_hbm.at[pl.ds(r0, RPT)], acc.at[pl.ds(r0, RPT)])

        ixr = (ixr0, ixr1, ixr2)
        ixc = (ixc0, ixc1, ixc2)
        si = (si0, si1, si2)
        buf = (bufa, bufb)
        sg = (sga, sgb)
        ss = (ssa, ssb)

        def issue_idx(s, q):
            pltpu.async_copy(row_hbm.at[cid, sid, s], ixr[q], si[q])
            pltpu.async_copy(col_hbm.at[sid, s], ixc[q], si[q])

        def wait_idx(s, q):
            pltpu.make_async_copy(row_hbm.at[cid, sid, s], ixr[q], si[q]).wait()
            pltpu.make_async_copy(col_hbm.at[sid, s], ixc[q], si[q]).wait()

        def issue_gathers(q, p):
            for b in range(SBF):
                pltpu.async_copy(g_hbm.at[ixr[q].at[b]], buf[p].at[b], sg[p])

        def wait_gathers(q, p):
            for b in range(SBF):
                pltpu.make_async_copy(g_hbm.at[ixr[q].at[b]], buf[p].at[b],
                                      sg[p]).wait()

        def issue_scatters(q, p):
            for b in range(SBF):
                pltpu.async_copy(buf[p].at[b], acc.at[ixc[q].at[b]],
                                 ss[p], add=True)

        def drain_scatters(q, p):
            for b in range(SBF):
                pltpu.make_async_copy(buf[p].at[b], acc.at[ixc[q].at[b]],
                                      ss[p]).wait()

        plsc.subcore_barrier()
        issue_idx(0, 0)
        issue_idx(1, 1)
        wait_idx(0, 0)
        issue_gathers(0, 0)

        def phase(s, q, p, drain=True, idx2=True, gnext=True):
            wait_gathers(q, p)
            issue_scatters(q, p)
            if drain:
                drain_scatters((q + 2) % 3, 1 - p)
            if idx2:
                issue_idx(s + 2, (q + 2) % 3)
            if gnext:
                wait_idx(s + 1, (q + 1) % 3)
                issue_gathers((q + 1) % 3, 1 - p)

        phase(0, 0, 0, drain=False)

        def body6(k_, carry):
            s0 = 6 * k_ + 1
            phase(s0 + 0, 1, 1)
            phase(s0 + 1, 2, 0)
            phase(s0 + 2, 0, 1)
            phase(s0 + 3, 1, 0)
            phase(s0 + 4, 2, 1)
            phase(s0 + 5, 0, 0)
            return carry

        lax.fori_loop(0, N6, body6, 0)
        for s in range(6 * N6 + 1, NSBF):
            phase(s, s % 3, s % 2, idx2=(s + 2 < NSBF), gnext=(s + 1 < NSBF))
        drain_scatters((NSBF - 1) % 3, (NSBF - 1) % 2)
        plsc.subcore_barrier()
        r1 = sid * RPN
        pltpu.sync_copy(acc.at[pl.ds(r1, RPN)],
                        out_hbm.at[pl.ds(r1, RPN), pl.ds(cid * FH, FH)])

    return k


@functools.partial(
    pl.kernel,
    out_type=jax.ShapeDtypeStruct((NC, NP), jnp.float32),
    mesh=_MESH,
    scratch_types=[
        pltpu.VMEM((NSB // NC, SB, CH), jnp.int32),
        pltpu.VMEM((CH,), jnp.float32),
        pltpu.VMEM_SHARED((NP,), jnp.float32),
    ],
)
def _deg_kernel(col_hbm, zero_hbm, out_hbm, idx_c, ones_v, acc):
    cid = lax.axis_index("c")
    sid = lax.axis_index("s")
    r0 = sid * RPT
    pltpu.sync_copy(zero_hbm.at[pl.ds(r0, RPT)], acc.at[pl.ds(r0, RPT)])
    pltpu.sync_copy(col_hbm.at[sid, pl.ds(cid * (NSB // NC), NSB // NC)], idx_c)
    for i in range(CH // 16):
        ones_v[pl.ds(i * 16, 16)] = jnp.ones((16,), jnp.float32)
    plsc.subcore_barrier()

    def body(j, carry):
        for b in range(SB):
            pltpu.sync_copy(ones_v, acc.at[idx_c.at[j, b]], add=True)
        return carry

    lax.fori_loop(0, NSB // NC, body, 0)
    plsc.subcore_barrier()
    pltpu.sync_copy(acc.at[pl.ds(r0, RPT)], out_hbm.at[cid, pl.ds(r0, RPT)])


BR = 2000
GRID = (N // BR,)


def _row(f):
    return pl.BlockSpec((BR, f), lambda i: (i, 0))


def _full(a, b):
    return pl.BlockSpec((a, b), lambda i: (0, 0))


def _f32(*shape):
    return jax.ShapeDtypeStruct(shape, jnp.float32)


def _k1_body(deg0, deg1, x, w, dinv_o, g_o):
    dinv = lax.rsqrt(deg0[...] + deg1[...] + 1.0)
    dinv_o[...] = dinv
    g_o[...] = dinv * jnp.dot(x[...], w[...], preferred_element_type=jnp.float32)


_k1 = pl.pallas_call(
    _k1_body,
    grid=GRID,
    in_specs=[_row(1), _row(1), _row(128), _full(128, 64)],
    out_specs=[_row(1), _row(64)],
    out_shape=[_f32(N, 1), _f32(N, 64)],
)


def _k2_body(s, g, dinv, b, w, g2_o):
    dv = dinv[...]
    h = jnp.maximum(dv * (s[...] + g[...]) + b[...], 0.0)
    g2_o[...] = dv * jnp.dot(h, w[...], preferred_element_type=jnp.float32)


_k2 = pl.pallas_call(
    _k2_body,
    grid=GRID,
    in_specs=[_row(64), _row(64), _row(1), _full(1, 64), _full(64, 32)],
    out_specs=[_row(32)],
    out_shape=[_f32(N, 32)],
)


def _k3_body(s, g, dinv, b, wmu, bmu, wlv, blv, wd1, mu_o, lv_o, g3_o):
    dv = dinv[...]
    h2 = dv * (s[...] + g[...]) + b[...]
    mu = jnp.dot(h2, wmu[...], preferred_element_type=jnp.float32) + bmu[...]
    lv = jnp.dot(h2, wlv[...], preferred_element_type=jnp.float32) + blv[...]
    mu_o[...] = mu
    lv_o[...] = lv
    g3_o[...] = dv * jnp.dot(mu, wd1[...], preferred_element_type=jnp.float32)


_k3 = pl.pallas_call(
    _k3_body,
    grid=GRID,
    in_specs=[
        _row(32), _row(32), _row(1), _full(1, 32),
        _full(32, 32), _full(1, 32), _full(32, 32), _full(1, 32), _full(32, 128),
    ],
    out_specs=[_row(32), _row(32), _row(128)],
    out_shape=[_f32(N, 32), _f32(N, 32), _f32(N, 128)],
)


def _k4_body(s, g, dinv, b, w, g4_o):
    dv = dinv[...]
    d = jnp.maximum(dv * (s[...] + g[...]) + b[...], 0.0)
    g4_o[...] = dv * jnp.dot(d, w[...], preferred_element_type=jnp.float32)


_k4 = pl.pallas_call(
    _k4_body,
    grid=GRID,
    in_specs=[_row(128), _row(128), _row(1), _full(1, 128), _full(128, 128)],
    out_specs=[_row(128)],
    out_shape=[_f32(N, 128)],
)


def _k5_body(s, g, dinv, b, recon_o):
    recon_o[...] = dinv[...] * (s[...] + g[...]) + b[...]


_k5 = pl.pallas_call(
    _k5_body,
    grid=GRID,
    in_specs=[_row(128), _row(128), _row(1), _full(1, 128)],
    out_specs=[_row(128)],
    out_shape=[_f32(N, 128)],
)


def _twin(g):
    return g.reshape(2 * N, g.shape[1] // 2)


def kernel(x, edge_index, W_e1, b_e1, W_e2, b_e2, W_mu, b_mu, W_lv, b_lv,
           W_d1, b_d1, W_d2, b_d2):
    ei = edge_index.astype(jnp.int32)
    row = ei[0].reshape(NS, NSB, SB, CH)
    rowx = jnp.stack([2 * row, 2 * row + 1])
    colx = ei[1].reshape(NS, NSB, SB, CH)

    degp = _deg_kernel(colx, jnp.zeros((NP,), jnp.float32))
    deg0 = degp[0, :N].reshape(N, 1)
    deg1 = degp[1, :N].reshape(N, 1)

    dinv, g1 = _k1(deg0, deg1, x, W_e1)

    rowx10 = rowx.reshape(NC, NS, NSB // 2, 2 * SB, CH)
    colx10 = colx.reshape(NS, NSB // 2, 2 * SB, CH)

    s1 = _make_edge_scatter(32)(_twin(g1), rowx10, colx10,
                                jnp.zeros((NP, 32), jnp.float32))
    (g2,) = _k2(s1, g1, dinv, b_e1.reshape(1, 64), W_e2)

    s2 = _make_edge_scatter(16)(_twin(g2), rowx10, colx10,
                                jnp.zeros((NP, 16), jnp.float32))
    mu, logvar, g3 = _k3(
        s2, g2, dinv, b_e2.reshape(1, 32),
        W_mu, b_mu.reshape(1, 32), W_lv, b_lv.reshape(1, 32), W_d1,
    )

    s3 = _make_edge_scatter(64)(_twin(g3), rowx, colx,
                                jnp.zeros((NP, 64), jnp.float32))
    (g4,) = _k4(s3, g3, dinv, b_d1.reshape(1, 128), W_d2)

    s4 = _make_edge_scatter(64)(_twin(g4), rowx, colx,
                                jnp.zeros((NP, 64), jnp.float32))
    (recon,) = _k5(s4, g4, dinv, b_d2.reshape(1, 128))

    return recon, mu, logvar


# speedup vs baseline: 30.1633x; 1.0098x over previous
import functools

import jax
import jax.numpy as jnp
from jax import lax
from jax.experimental import pallas as pl
from jax.experimental.pallas import tpu as pltpu
from jax.experimental.pallas import tpu_sc as plsc

N = 10000
NP = 10240
E = 320000
NC = 2
NS = 16
NW = NC * NS
CH = 80
ET = E // NS
NCHUNK = ET // CH
SB = 5
NSB = NCHUNK // SB
DNCHUNK = NCHUNK // NC
RPT = NP // NS
RPN = N // NS

_MESH = plsc.VectorSubcoreMesh(core_axis_name="c", subcore_axis_name="s")


@functools.cache
def _make_edge_scatter(FH):
    SBF = 25 if FH <= 16 else (10 if FH <= 32 else SB)
    NSBF = NCHUNK // SBF
    N6 = (NSBF - 3) // 6

    @functools.partial(
        pl.kernel,
        out_type=jax.ShapeDtypeStruct((N, 2 * FH), jnp.float32),
        mesh=_MESH,
        scratch_types=[
            *[pltpu.VMEM((SBF, CH), jnp.int32) for _ in range(3)],
            *[pltpu.VMEM((SBF, CH), jnp.int32) for _ in range(3)],
            pltpu.VMEM((SBF, CH, FH), jnp.float32),
            pltpu.VMEM((SBF, CH, FH), jnp.float32),
            pltpu.VMEM_SHARED((NP, FH), jnp.float32),
            *[pltpu.SemaphoreType.DMA for _ in range(7)],
        ],
        compiler_params=pltpu.CompilerParams(use_tc_tiling_on_sc=False),
    )
    def k(g_hbm, row_hbm, col_hbm, zero_hbm, out_hbm,
          ixr0, ixr1, ixr2, ixc0, ixc1, ixc2, bufa, bufb, acc,
          si0, si1, si2, sga, sgb, ssa, ssb):
        cid = lax.axis_index("c")
        sid = lax.axis_index("s")
        r0 = sid * RPT
        pltpu.sync_copy(zero_hbm.at[pl.ds(r0, RPT)], acc.at[pl.ds(r0, RPT)])

        ixr = (ixr0, ixr1, ixr2)
        ixc = (ixc0, ixc1, ixc2)
        si = (si0, si1, si2)
        buf = (bufa, bufb)
        sg = (sga, sgb)
        ss = (ssa, ssb)

        def issue_idx(s, q):
            pltpu.async_copy(row_hbm.at[cid, sid, s], ixr[q], si[q])
            pltpu.async_copy(col_hbm.at[sid, s], ixc[q], si[q])

        def wait_idx(s, q):
            pltpu.make_async_copy(row_hbm.at[cid, sid, s], ixr[q], si[q]).wait()
            pltpu.make_async_copy(col_hbm.at[sid, s], ixc[q], si[q]).wait()

        def issue_gathers(q, p):
            for b in range(SBF):
                pltpu.async_copy(g_hbm.at[ixr[q].at[b]], buf[p].at[b], sg[p])

        def wait_gathers(q, p):
            for b in range(SBF):
                pltpu.make_async_copy(g_hbm.at[ixr[q].at[b]], buf[p].at[b],
                                      sg[p]).wait()

        def issue_scatters(q, p):
            for b in range(SBF):
                pltpu.async_copy(buf[p].at[b], acc.at[ixc[q].at[b]],
                                 ss[p], add=True)

        def drain_scatters(q, p):
            for b in range(SBF):
                pltpu.make_async_copy(buf[p].at[b], acc.at[ixc[q].at[b]],
                                      ss[p]).wait()

        plsc.subcore_barrier()
        issue_idx(0, 0)
        issue_idx(1, 1)
        wait_idx(0, 0)
        issue_gathers(0, 0)

        def phase(s, q, p, drain=True, idx2=True, gnext=True):
            wait_gathers(q, p)
            issue_scatters(q, p)
            if drain:
                drain_scatters((q + 2) % 3, 1 - p)
            if idx2:
                issue_idx(s + 2, (q + 2) % 3)
            if gnext:
                wait_idx(s + 1, (q + 1) % 3)
                issue_gathers((q + 1) % 3, 1 - p)

        phase(0, 0, 0, drain=False)

        def body6(k_, carry):
            s0 = 6 * k_ + 1
            phase(s0 + 0, 1, 1)
            phase(s0 + 1, 2, 0)
            phase(s0 + 2, 0, 1)
            phase(s0 + 3, 1, 0)
            phase(s0 + 4, 2, 1)
            phase(s0 + 5, 0, 0)
            return carry

        lax.fori_loop(0, N6, body6, 0)
        for s in range(6 * N6 + 1, NSBF):
            phase(s, s % 3, s % 2, idx2=(s + 2 < NSBF), gnext=(s + 1 < NSBF))
        drain_scatters((NSBF - 1) % 3, (NSBF - 1) % 2)
        plsc.subcore_barrier()
        r1 = sid * RPN
        pltpu.sync_copy(acc.at[pl.ds(r1, RPN)],
                        out_hbm.at[pl.ds(r1, RPN), pl.ds(cid * FH, FH)])

    return k


@functools.partial(
    pl.kernel,
    out_type=jax.ShapeDtypeStruct((NC, NP), jnp.float32),
    mesh=_MESH,
    scratch_types=[
        pltpu.VMEM((NSB // NC, SB, CH), jnp.int32),
        pltpu.VMEM((CH,), jnp.float32),
        pltpu.VMEM_SHARED((NP,), jnp.float32),
    ],
)
def _deg_kernel(col_hbm, zero_hbm, out_hbm, idx_c, ones_v, acc):
    cid = lax.axis_index("c")
    sid = lax.axis_index("s")
    r0 = sid * RPT
    pltpu.sync_copy(zero_hbm.at[pl.ds(r0, RPT)], acc.at[pl.ds(r0, RPT)])
    pltpu.sync_copy(col_hbm.at[sid, pl.ds(cid * (NSB // NC), NSB // NC)], idx_c)
    for i in range(CH // 16):
        ones_v[pl.ds(i * 16, 16)] = jnp.ones((16,), jnp.float32)
    plsc.subcore_barrier()

    def body(j, carry):
        for b in range(SB):
            pltpu.sync_copy(ones_v, acc.at[idx_c.at[j, b]], add=True)
        return carry

    lax.fori_loop(0, NSB // NC, body, 0)
    plsc.subcore_barrier()
    pltpu.sync_copy(acc.at[pl.ds(r0, RPT)], out_hbm.at[cid, pl.ds(r0, RPT)])


BR = 2000
GRID = (N // BR,)


def _row(f):
    return pl.BlockSpec((BR, f), lambda i: (i, 0))


def _full(a, b):
    return pl.BlockSpec((a, b), lambda i: (0, 0))


def _f32(*shape):
    return jax.ShapeDtypeStruct(shape, jnp.float32)


def _k1_body(deg0, deg1, x, w, dinv_o, g_o):
    dinv = lax.rsqrt(deg0[...] + deg1[...] + 1.0)
    dinv_o[...] = dinv
    g_o[...] = dinv * jnp.dot(x[...], w[...], preferred_element_type=jnp.float32)


_k1 = pl.pallas_call(
    _k1_body,
    grid=GRID,
    in_specs=[_row(1), _row(1), _row(128), _full(128, 64)],
    out_specs=[_row(1), _row(64)],
    out_shape=[_f32(N, 1), _f32(N, 64)],
)


def _k2_body(s, g, dinv, b, w, g2_o):
    dv = dinv[...]
    h = jnp.maximum(dv * (s[...] + g[...]) + b[...], 0.0)
    g2_o[...] = dv * jnp.dot(h, w[...], preferred_element_type=jnp.float32)


_k2 = pl.pallas_call(
    _k2_body,
    grid=GRID,
    in_specs=[_row(64), _row(64), _row(1), _full(1, 64), _full(64, 32)],
    out_specs=[_row(32)],
    out_shape=[_f32(N, 32)],
)


def _k3_body(s, g, dinv, b, wmu, bmu, wlv, blv, wd1, mu_o, lv_o, g3_o):
    dv = dinv[...]
    h2 = dv * (s[...] + g[...]) + b[...]
    mu = jnp.dot(h2, wmu[...], preferred_element_type=jnp.float32) + bmu[...]
    lv = jnp.dot(h2, wlv[...], preferred_element_type=jnp.float32) + blv[...]
    mu_o[...] = mu
    lv_o[...] = lv
    g3_o[...] = dv * jnp.dot(mu, wd1[...], preferred_element_type=jnp.float32)


_k3 = pl.pallas_call(
    _k3_body,
    grid=GRID,
    in_specs=[
        _row(32), _row(32), _row(1), _full(1, 32),
        _full(32, 32), _full(1, 32), _full(32, 32), _full(1, 32), _full(32, 128),
    ],
    out_specs=[_row(32), _row(32), _row(128)],
    out_shape=[_f32(N, 32), _f32(N, 32), _f32(N, 128)],
)


def _k4_body(s, g, dinv, b, w, g4_o):
    dv = dinv[...]
    d = jnp.maximum(dv * (s[...] + g[...]) + b[...], 0.0)
    g4_o[...] = dv * jnp.dot(d, w[...], preferred_element_type=jnp.float32)


_k4 = pl.pallas_call(
    _k4_body,
    grid=GRID,
    in_specs=[_row(128), _row(128), _row(1), _full(1, 128), _full(128, 128)],
    out_specs=[_row(128)],
    out_shape=[_f32(N, 128)],
)


def _k5_body(s, g, dinv, b, recon_o):
    recon_o[...] = dinv[...] * (s[...] + g[...]) + b[...]


_k5 = pl.pallas_call(
    _k5_body,
    grid=GRID,
    in_specs=[_row(128), _row(128), _row(1), _full(1, 128)],
    out_specs=[_row(128)],
    out_shape=[_f32(N, 128)],
)


def _twin(g):
    return g.reshape(2 * N, g.shape[1] // 2)


def kernel(x, edge_index, W_e1, b_e1, W_e2, b_e2, W_mu, b_mu, W_lv, b_lv,
           W_d1, b_d1, W_d2, b_d2):
    ei = edge_index.astype(jnp.int32)
    row = ei[0].reshape(NS, NSB, SB, CH)
    rowx = jnp.stack([2 * row, 2 * row + 1])
    colx = ei[1].reshape(NS, NSB, SB, CH)

    degp = _deg_kernel(colx, jnp.zeros((NP,), jnp.float32))
    deg0 = degp[0, :N].reshape(N, 1)
    deg1 = degp[1, :N].reshape(N, 1)

    dinv, g1 = _k1(deg0, deg1, x, W_e1)

    rowx10 = rowx.reshape(NC, NS, NSB // 2, 2 * SB, CH)
    colx10 = colx.reshape(NS, NSB // 2, 2 * SB, CH)

    s1 = _make_edge_scatter(32)(_twin(g1), rowx10, colx10,
                                jnp.zeros((NP, 32), jnp.float32))
    (g2,) = _k2(s1, g1, dinv, b_e1.reshape(1, 64), W_e2)

    rowx25 = rowx.reshape(NC, NS, NSB // 5, 5 * SB, CH)
    colx25 = colx.reshape(NS, NSB // 5, 5 * SB, CH)
    s2 = _make_edge_scatter(16)(_twin(g2), rowx25, colx25,
                                jnp.zeros((NP, 16), jnp.float32))
    mu, logvar, g3 = _k3(
        s2, g2, dinv, b_e2.reshape(1, 32),
        W_mu, b_mu.reshape(1, 32), W_lv, b_lv.reshape(1, 32), W_d1,
    )

    s3 = _make_edge_scatter(64)(_twin(g3), rowx, colx,
                                jnp.zeros((NP, 64), jnp.float32))
    (g4,) = _k4(s3, g3, dinv, b_d1.reshape(1, 128), W_d2)

    s4 = _make_edge_scatter(64)(_twin(g4), rowx, colx,
                                jnp.zeros((NP, 64), jnp.float32))
    (recon,) = _k5(s4, g4, dinv, b_d2.reshape(1, 128))

    return recon, mu, logvar
